# baseline (speedup 1.0000x reference)
import functools

import jax
import jax.numpy as jnp
from jax import lax
from jax.experimental import pallas as pl
from jax.experimental.pallas import tpu as pltpu
from jax.experimental.pallas import tpu_sc as plsc

D = 32
PACK = 4
GW = PACK * D
G = 128
OG = G // PACK
NBUF = 4


@functools.cache
def _make(total_rows: int):
    info = plsc.get_sparse_core_info()
    nc, ns = info.num_cores, info.num_subcores
    nw = nc * ns
    assert total_rows % (nw * G) == 0
    n_groups = total_rows // (nw * G)
    assert n_groups % NBUF == 0
    total_q = total_rows // PACK

    mesh = plsc.VectorSubcoreMesh(core_axis_name="c", subcore_axis_name="s")

    @functools.partial(
        pl.kernel,
        mesh=mesh,
        out_type=jax.ShapeDtypeStruct((total_q, GW), jnp.float32),
        scratch_types=[
            pltpu.VMEM((n_groups, G), jnp.int32),
            pltpu.VMEM((NBUF, G), jnp.int32),
            *[pltpu.VMEM((G, GW), jnp.float32) for _ in range(NBUF)],
            *[pltpu.VMEM((OG, GW), jnp.float32) for _ in range(NBUF)],
            *[pltpu.SemaphoreType.DMA for _ in range(2 * NBUF)],
        ],
    )
    def gather_kernel(table_q, idx, out_q, idx_v, qidx, *rest):
        rows = rest[:NBUF]
        obuf = rest[NBUF:2 * NBUF]
        gsem = rest[2 * NBUF:3 * NBUF]
        osem = rest[3 * NBUF:]
        wid = lax.axis_index("s") * nc + lax.axis_index("c")
        base_g = wid * n_groups
        base_q = base_g * OG

        pltpu.sync_copy(idx.at[pl.ds(base_g, n_groups)], idx_v)

        def fire(j, b):
            for m in range(G // 16):
                ids = idx_v[j, pl.ds(16 * m, 16)]
                qidx[b, pl.ds(16 * m, 16)] = ids >> 2
            pltpu.async_copy(table_q.at[qidx.at[b]], rows[b], gsem[b])

        for b in range(NBUF):
            fire(b, b)

        def step(j0, carry):
            for b in range(NBUF):
                j = j0 * NBUF + b
                pltpu.make_async_copy(
                    table_q.at[qidx.at[b]], rows[b], gsem[b]).wait()

                @pl.when(j >= NBUF)
                def _():
                    pltpu.make_async_copy(
                        obuf[b], out_q.at[pl.ds(base_q, OG)], osem[b]).wait()

                def extract(m, c):
                    ids = idx_v[j, pl.ds(16 * m, 16)]
                    for l in range(16):
                        p = 16 * m + l
                        scol = (ids[l] & 3) * D
                        orow = 4 * m + l // 4
                        ocol = (l & 3) * D
                        for h in range(D // 16):
                            v = rows[b][p, pl.ds(scol + 16 * h, 16)]
                            obuf[b][orow, pl.ds(ocol + 16 * h, 16)] = v
                    return c

                lax.fori_loop(0, G // 16, extract, 0)
                pltpu.async_copy(
                    obuf[b], out_q.at[pl.ds(base_q + j * OG, OG)], osem[b])
                nj = j + NBUF

                @pl.when(nj < n_groups)
                def _():
                    fire(nj, b)
            return carry

        lax.fori_loop(0, n_groups // NBUF, step, 0)
        for b in range(NBUF):
            pltpu.make_async_copy(
                obuf[b], out_q.at[pl.ds(base_q, OG)], osem[b]).wait()

    return gather_kernel


def kernel(token_ids, time_embeddings_param):
    b, t = token_ids.shape
    total = b * t
    idx2d = token_ids.astype(jnp.int32).reshape(total // G, G)
    table_q = time_embeddings_param.reshape(-1, GW)
    out = _make(total)(table_q, idx2d)
    return out.reshape(b, t, D)


# speedup vs baseline: 1.3078x; 1.3078x over previous
import functools

import jax
import jax.numpy as jnp
from jax import lax
from jax.experimental import pallas as pl
from jax.experimental.pallas import tpu as pltpu
from jax.experimental.pallas import tpu_sc as plsc

D = 32
NB = 4
FLY = 3


@functools.cache
def _make(b, t, v):
    info = plsc.get_sparse_core_info()
    nc, ns = info.num_cores, info.num_subcores
    nw = nc * ns
    assert b % nw == 0
    bat_w = b // nw
    ids_w = bat_w * t
    nfull = t // 16
    tail = t - 16 * nfull

    mesh = plsc.VectorSubcoreMesh(core_axis_name="c", subcore_axis_name="s")

    @functools.partial(
        pl.kernel,
        mesh=mesh,
        out_type=jax.ShapeDtypeStruct((b, t, D), jnp.float32),
        compiler_params=pltpu.CompilerParams(use_tc_tiling_on_sc=True),
        scratch_types=[
            pltpu.VMEM((ids_w,), jnp.int32),
            pltpu.VMEM((NB, t, D), jnp.float32),
            *[pltpu.SemaphoreType.DMA for _ in range(2 * NB)],
        ],
    )
    def gather_kernel(table, idx, out, ids_v, obuf, *sems):
        gsem = sems[:NB]
        osem = sems[NB:]
        wid = lax.axis_index("s") * nc + lax.axis_index("c")

        pltpu.sync_copy(idx.at[pl.ds(wid * ids_w, ids_w)], ids_v)

        def fire16(i, m, s, n):
            ids = ids_v[pl.ds(t * i + 16 * m, 16)]
            for l in range(n):
                pltpu.async_copy(
                    table.at[pl.ds(ids[l], 1)],
                    obuf.at[s, pl.ds(16 * m + l, 1)], gsem[s])

        def fire(i, s):
            def body(m, c):
                fire16(i, m, s, 16)
                return c

            lax.fori_loop(0, nfull, body, 0)
            if tail:
                fire16(i, nfull, s, tail)

        def drain_gather(s):
            pltpu.make_async_copy(
                table.at[pl.ds(0, t)], obuf.at[s], gsem[s]).wait()

        for s in range(FLY):
            fire(s, s)

        def step(i, carry):
            for u in range(NB):
                j = i * NB + u
                s = u % NB
                sn = (u + FLY) % NB
                nj = j + FLY

                @pl.when(nj < bat_w)
                def _():
                    @pl.when(nj >= NB)
                    def _():
                        pltpu.make_async_copy(
                            obuf.at[sn], out.at[wid * bat_w], osem[sn]).wait()
                    fire(nj, sn)

                drain_gather(s)
                pltpu.async_copy(obuf.at[s], out.at[wid * bat_w + j], osem[s])
            return carry

        lax.fori_loop(0, bat_w // NB, step, 0)
        for s in range(NB):
            pltpu.make_async_copy(
                obuf.at[s], out.at[wid * bat_w], osem[s]).wait()

    return gather_kernel


def kernel(token_ids, time_embeddings_param):
    b, t = token_ids.shape
    idx1 = token_ids.astype(jnp.int32).reshape(-1)
    return _make(b, t, time_embeddings_param.shape[0])(
        time_embeddings_param, idx1)
